# baseline (speedup 1.0000x reference)
import functools

import jax
import jax.numpy as jnp
from jax import lax
from jax.experimental import pallas as pl
from jax.experimental.pallas import tpu as pltpu
from jax.experimental.pallas import tpu_sc as plsc

NUM_RELATIONS = 1000000
EMBEDDING_DIM = 32
BATCH = 4096
FIELDS = 26

_info = plsc.get_sparse_core_info()
_NC, _NS = _info.num_cores, _info.num_subcores
_NW = _NC * _NS
_NROWS = FIELDS * EMBEDDING_DIM
_RPW = _NROWS // _NW
_NBUF = 4


@functools.partial(
    pl.kernel,
    out_type=jax.ShapeDtypeStruct((FIELDS, EMBEDDING_DIM, BATCH), jnp.float32),
    mesh=plsc.VectorSubcoreMesh(core_axis_name="c", subcore_axis_name="s"),
    scratch_types=[
        pltpu.VMEM((_NBUF, BATCH), jnp.int32),
        pltpu.VMEM((_NBUF, BATCH), jnp.float32),
        pltpu.SemaphoreType.DMA((_NBUF,)),
        pltpu.SemaphoreType.DMA((_NBUF,)),
        pltpu.SemaphoreType.DMA((_NBUF,)),
    ],
    compiler_params=pltpu.CompilerParams(use_tc_tiling_on_sc=False),
)
def _gather_kernel(tab_hbm, ids_hbm, out_hbm, idx_v, row_v, isems, gsems, wsems):
    wid = lax.axis_index("s") * _NC + lax.axis_index("c")
    r0 = wid * _RPW

    ih = [None] * _RPW
    gh = [None] * _RPW
    wh = [None] * _RPW

    def row_fd(k):
        r = r0 + k
        return r // EMBEDDING_DIM, r % EMBEDDING_DIM

    for t in range(_RPW + 2):
        k = t
        if k < _RPW:
            b = k % _NBUF
            if k >= _NBUF:
                wh[k - _NBUF].wait()
            f, _ = row_fd(k)
            ih[k] = pltpu.async_copy(ids_hbm.at[f], idx_v.at[b], isems.at[b])
        k = t - 1
        if 0 <= k < _RPW:
            b = k % _NBUF
            ih[k].wait()
            _, d = row_fd(k)
            gh[k] = pltpu.async_copy(
                tab_hbm.at[d].at[idx_v.at[b]], row_v.at[b], gsems.at[b]
            )
        k = t - 2
        if 0 <= k < _RPW:
            b = k % _NBUF
            gh[k].wait()
            f, d = row_fd(k)
            wh[k] = pltpu.async_copy(row_v.at[b], out_hbm.at[f, d], wsems.at[b])
    for k in range(_RPW - _NBUF, _RPW):
        wh[k].wait()


@jax.jit
def kernel(relation_ids, embedding_table):
    tab_t = embedding_table.T
    ids_t = relation_ids.T.astype(jnp.int32)
    out = _gather_kernel(tab_t, ids_t)
    return out.transpose(2, 0, 1)
